# baseline (speedup 1.0000x reference)
import functools

import jax
import jax.numpy as jnp
from jax import lax
from jax.experimental import pallas as pl
from jax.experimental.pallas import tpu as pltpu
from jax.experimental.pallas import tpu_sc as plsc

N = 10000
E = 320000
D_IN = 128
H = 32
G = 64
D_OUT = 128

NC = 2
NS = 16
NW = NC * NS

CHUNK = 128
OPS_PER_TILE = 80
REAL_CHUNKS = E // CHUNK
LAST_REAL = REAL_CHUNKS - (NW - 1) * OPS_PER_TILE
PAD_OPS = OPS_PER_TILE - LAST_REAL
NP = 10240
ROWS_PER_TILE = NP // NS


_sc_mesh = plsc.VectorSubcoreMesh(core_axis_name="c", subcore_axis_name="s")


@functools.partial(
    pl.kernel,
    out_type=jax.ShapeDtypeStruct((NC * NP, H), jnp.float32),
    mesh=_sc_mesh,
    scratch_types=[
        pltpu.VMEM((OPS_PER_TILE, CHUNK), jnp.int32),
        pltpu.VMEM((OPS_PER_TILE, CHUNK), jnp.int32),
        pltpu.VMEM((CHUNK, H), jnp.float32),
        pltpu.VMEM((CHUNK, H), jnp.float32),
        pltpu.VMEM((CHUNK, H), jnp.float32),
        pltpu.VMEM((CHUNK, H), jnp.float32),
        pltpu.VMEM_SHARED((NP, H), jnp.float32),
        pltpu.VMEM_SHARED((NP, H), jnp.float32),
        pltpu.SemaphoreType.DMA,
        pltpu.SemaphoreType.DMA,
        pltpu.SemaphoreType.DMA,
        pltpu.SemaphoreType.DMA,
        pltpu.SemaphoreType.DMA,
        pltpu.SemaphoreType.DMA,
        pltpu.SemaphoreType.DMA,
        pltpu.SemaphoreType.DMA,
    ],
    compiler_params=pltpu.CompilerParams(use_tc_tiling_on_sc=False),
)
def _sc_segment_sum(z_hbm, src_hbm, dst_hbm, zeros_hbm, pad_hbm, out_hbm,
                    sidx, didx, rb0, rb1, rb2, rb3, zsp, acc,
                    g0, g1, g2, g3, s0, s1, s2, s3):
    c = lax.axis_index("c")
    s = lax.axis_index("s")
    wid = c * NS + s
    r0 = s * ROWS_PER_TILE

    st_z = pltpu.make_async_copy(z_hbm.at[pl.ds(r0, ROWS_PER_TILE)],
                                 zsp.at[pl.ds(r0, ROWS_PER_TILE)], g2)
    st_acc = pltpu.make_async_copy(zeros_hbm.at[pl.ds(r0, ROWS_PER_TILE)],
                                   acc.at[pl.ds(r0, ROWS_PER_TILE)], g3)
    st_z.start()
    st_acc.start()

    @pl.when(wid < NW - 1)
    def _():
        a = pltpu.make_async_copy(
            src_hbm.at[pl.ds(wid * OPS_PER_TILE, OPS_PER_TILE)], sidx, g0)
        b = pltpu.make_async_copy(
            dst_hbm.at[pl.ds(wid * OPS_PER_TILE, OPS_PER_TILE)], didx, g1)
        a.start()
        b.start()
        a.wait()
        b.wait()

    @pl.when(wid == NW - 1)
    def _():
        a = pltpu.make_async_copy(
            src_hbm.at[pl.ds((NW - 1) * OPS_PER_TILE, LAST_REAL)],
            sidx.at[pl.ds(0, LAST_REAL)], g0)
        b = pltpu.make_async_copy(
            dst_hbm.at[pl.ds((NW - 1) * OPS_PER_TILE, LAST_REAL)],
            didx.at[pl.ds(0, LAST_REAL)], g1)
        cpad = pltpu.make_async_copy(
            pad_hbm, sidx.at[pl.ds(LAST_REAL, PAD_OPS)], s0)
        dpad = pltpu.make_async_copy(
            pad_hbm, didx.at[pl.ds(LAST_REAL, PAD_OPS)], s1)
        a.start()
        b.start()
        cpad.start()
        dpad.start()
        a.wait()
        b.wait()
        cpad.wait()
        dpad.wait()

    st_z.wait()
    st_acc.wait()
    plsc.subcore_barrier()

    bufs = (rb0, rb1, rb2, rb3)
    gsem = (g0, g1, g2, g3)
    ssem = (s0, s1, s2, s3)

    def _drain(j, sems):
        pltpu.make_async_copy(z_hbm.at[pl.ds(0, CHUNK)], bufs[j],
                              sems[j]).wait()

    def _gat(k, j):
        pltpu.async_copy(zsp.at[sidx.at[k]], bufs[j], gsem[j])

    def _scat(k, j):
        pltpu.async_copy(bufs[j], acc.at[didx.at[k]], ssem[j], add=True)

    def _step(k, j, lookahead):
        _drain(j, gsem)
        _scat(k, j)
        if lookahead:
            _drain((j + 2) % 4, ssem)
            _gat(k + 2, (j + 2) % 4)

    _gat(0, 0)
    _gat(1, 1)
    _drain(0, gsem); _scat(0, 0); _gat(2, 2)
    _drain(1, gsem); _scat(1, 1); _gat(3, 3)
    _step(2, 2, True)
    _step(3, 3, True)

    def body(b, carry):
        k = 4 * b
        _step(k, 0, True)
        _step(k + 1, 1, True)
        _step(k + 2, 2, True)
        _step(k + 3, 3, True)
        return carry

    lax.fori_loop(1, OPS_PER_TILE // 4 - 1, body, 0)

    _step(OPS_PER_TILE - 4, 0, True)
    _step(OPS_PER_TILE - 3, 1, True)
    _drain(2, gsem); _scat(OPS_PER_TILE - 2, 2)
    _drain(3, gsem); _scat(OPS_PER_TILE - 1, 3)
    _drain(0, ssem)
    _drain(1, ssem)
    _drain(2, ssem)
    _drain(3, ssem)
    plsc.subcore_barrier()

    pltpu.sync_copy(acc.at[pl.ds(r0, ROWS_PER_TILE)],
                    out_hbm.at[pl.ds(c * NP + r0, ROWS_PER_TILE)])


NPR = NP // 4
NRR = N // 4
PW = 4 * H


def _fold4(v):
    return v[:, 0:H] + v[:, H:2 * H] + v[:, 2 * H:3 * H] + v[:, 3 * H:4 * H]


def _tile4(v):
    return jnp.concatenate([v, v, v, v], axis=1)


def _proj_body(x_ref, w_ref, o_ref):
    o_ref[:NRR, :] = jnp.dot(x_ref[...], w_ref[...],
                             preferred_element_type=jnp.float32)
    o_ref[NRR:, :] = jnp.zeros((NPR - NRR, PW), jnp.float32)


def _proj(x4, w1bd):
    return pl.pallas_call(
        _proj_body,
        out_shape=jax.ShapeDtypeStruct((NPR, PW), jnp.float32),
    )(x4, w1bd)


def _bn_tail(pre, gamma_ref, beta_ref, w2bd_ref, b2_ref):
    s1 = _fold4(jnp.sum(pre, axis=0, keepdims=True)) / N
    s2 = _fold4(jnp.sum(pre * pre, axis=0, keepdims=True)) / N
    mean = _tile4(s1)
    var = _tile4(s2 - s1 * s1)
    hn = (pre - mean) * lax.rsqrt(var + 1e-5) * gamma_ref[...] + beta_ref[...]
    hn = jnp.maximum(hn, 0.0)
    h2 = jnp.dot(hn, w2bd_ref[...],
                 preferred_element_type=jnp.float32) + b2_ref[...]
    return jnp.maximum(h2, 0.0)


def _mid_body(z_ref, part_ref, b1_ref, gamma_ref, beta_ref, w2bd_ref, b2_ref,
              wnbd_ref, o_ref):
    pre = (z_ref[:NRR, :] + part_ref[:NRR, :]
           + part_ref[NPR:NPR + NRR, :] + b1_ref[...])
    h2 = _bn_tail(pre, gamma_ref, beta_ref, w2bd_ref, b2_ref)
    o_ref[:NRR, :] = jnp.dot(h2, wnbd_ref[...],
                             preferred_element_type=jnp.float32)
    o_ref[NRR:, :] = jnp.zeros((NPR - NRR, PW), jnp.float32)


def _mid(z, partp, pp):
    return pl.pallas_call(
        _mid_body,
        out_shape=jax.ShapeDtypeStruct((NPR, PW), jnp.float32),
    )(z, partp, pp["b1"], pp["gamma"], pp["beta"], pp["W2bd"], pp["b2"],
      pp["Wnbd"])


def _last_body(z_ref, part_ref, b1_ref, gamma_ref, beta_ref, w2bd_ref, b2_ref,
               hw1bd_ref, batch_ref, hb1_ref, hw2_ref, hb2_ref, o_ref):
    pre = (z_ref[:NRR, :] + part_ref[:NRR, :]
           + part_ref[NPR:NPR + NRR, :] + b1_ref[...])
    h2 = _bn_tail(pre, gamma_ref, beta_ref, w2bd_ref, b2_ref)
    hz = jnp.dot(h2, hw1bd_ref[...], preferred_element_type=jnp.float32)
    gids = lax.broadcasted_iota(jnp.int32, (G, NRR), 0)
    pooled = jnp.zeros((G, H), jnp.float32)
    for j in range(4):
        onehot = jnp.where(gids == batch_ref[j:j + 1, :NRR], 1.0, 0.0)
        pooled = pooled + jnp.dot(onehot, hz[:, j * H:(j + 1) * H],
                                  preferred_element_type=jnp.float32)
    hh = jnp.maximum(pooled + hb1_ref[...], 0.0)
    o_ref[...] = jnp.dot(hh, hw2_ref[...],
                         preferred_element_type=jnp.float32) + hb2_ref[...]


def _last(z, partp, pp, head, batchp):
    return pl.pallas_call(
        _last_body,
        out_shape=jax.ShapeDtypeStruct((G, D_OUT), jnp.float32),
    )(z, partp, pp["b1"], pp["gamma"], pp["beta"], pp["W2bd"], pp["b2"],
      jnp.kron(jnp.eye(4, dtype=jnp.float32), head["W1"]), batchp,
      head["b1"].reshape(1, H), head["W2"], head["b2"].reshape(1, D_OUT))


def _pack_params(p, w_next):
    eye4 = jnp.eye(4, dtype=jnp.float32)
    return {
        "b1": _tile4(p["b1"].reshape(1, H)),
        "gamma": _tile4(p["gamma"].reshape(1, H)),
        "beta": _tile4(p["beta"].reshape(1, H)),
        "W2bd": jnp.kron(eye4, p["W2"]),
        "b2": _tile4(p["b2"].reshape(1, H)),
        "Wnbd": None if w_next is None else jnp.kron(eye4, w_next),
    }


def kernel(x, edge_index, batch, params):
    src = edge_index[0].reshape(REAL_CHUNKS, CHUNK)
    dst = edge_index[1].reshape(REAL_CHUNKS, CHUNK)
    convs = params["convs"]
    zeros = jnp.zeros((NP, H), jnp.float32)
    pad_idx = jnp.full((PAD_OPS, CHUNK), N, jnp.int32)
    w1bd0 = jnp.kron(jnp.eye(4, dtype=jnp.float32), convs[0]["W1"])
    batchp = jnp.concatenate(
        [batch, jnp.full((NP - N,), G, jnp.int32)]).reshape(NPR, 4).T
    z = _proj(x.reshape(NRR, 4 * D_IN), w1bd0)
    for l in range(5):
        part = _sc_segment_sum(z.reshape(NP, H), src, dst, zeros, pad_idx)
        partp = part.reshape(2 * NPR, PW)
        if l < 4:
            z = _mid(z, partp, _pack_params(convs[l], convs[l + 1]["W1"]))
        else:
            out = _last(z, partp, _pack_params(convs[4], None),
                        params["head"], batchp)
    return out


# speedup vs baseline: 1.0050x; 1.0050x over previous
import functools

import jax
import jax.numpy as jnp
from jax import lax
from jax.experimental import pallas as pl
from jax.experimental.pallas import tpu as pltpu
from jax.experimental.pallas import tpu_sc as plsc

N = 10000
E = 320000
D_IN = 128
H = 32
G = 64
D_OUT = 128

NC = 2
NS = 16
NW = NC * NS

CHUNK = 128
OPS_PER_TILE = 80
REAL_CHUNKS = E // CHUNK
LAST_REAL = REAL_CHUNKS - (NW - 1) * OPS_PER_TILE
PAD_OPS = OPS_PER_TILE - LAST_REAL
NP = 10240
ROWS_PER_TILE = NP // NS


_sc_mesh = plsc.VectorSubcoreMesh(core_axis_name="c", subcore_axis_name="s")


@functools.partial(
    pl.kernel,
    out_type=jax.ShapeDtypeStruct((NC * NP, H), jnp.float32),
    mesh=_sc_mesh,
    scratch_types=[
        pltpu.VMEM((OPS_PER_TILE, CHUNK), jnp.int32),
        pltpu.VMEM((OPS_PER_TILE, CHUNK), jnp.int32),
        pltpu.VMEM((CHUNK, H), jnp.float32),
        pltpu.VMEM((CHUNK, H), jnp.float32),
        pltpu.VMEM((CHUNK, H), jnp.float32),
        pltpu.VMEM((CHUNK, H), jnp.float32),
        pltpu.VMEM_SHARED((NP, H), jnp.float32),
        pltpu.VMEM_SHARED((NP, H), jnp.float32),
        pltpu.SemaphoreType.DMA,
        pltpu.SemaphoreType.DMA,
        pltpu.SemaphoreType.DMA,
        pltpu.SemaphoreType.DMA,
        pltpu.SemaphoreType.DMA,
        pltpu.SemaphoreType.DMA,
        pltpu.SemaphoreType.DMA,
        pltpu.SemaphoreType.DMA,
    ],
    compiler_params=pltpu.CompilerParams(use_tc_tiling_on_sc=False),
)
def _sc_segment_sum(z_hbm, src_hbm, dst_hbm, zeros_hbm, pad_hbm, out_hbm,
                    sidx, didx, rb0, rb1, rb2, rb3, zsp, acc,
                    g0, g1, g2, g3, s0, s1, s2, s3):
    c = lax.axis_index("c")
    s = lax.axis_index("s")
    wid = s * NC + c
    r0 = s * ROWS_PER_TILE

    st_z = pltpu.make_async_copy(z_hbm.at[pl.ds(r0, ROWS_PER_TILE)],
                                 zsp.at[pl.ds(r0, ROWS_PER_TILE)], g2)
    st_acc = pltpu.make_async_copy(zeros_hbm.at[pl.ds(r0, ROWS_PER_TILE)],
                                   acc.at[pl.ds(r0, ROWS_PER_TILE)], g3)
    st_z.start()
    st_acc.start()

    @pl.when(wid < NW - 1)
    def _():
        a = pltpu.make_async_copy(
            src_hbm.at[pl.ds(wid * OPS_PER_TILE, OPS_PER_TILE)], sidx, g0)
        b = pltpu.make_async_copy(
            dst_hbm.at[pl.ds(wid * OPS_PER_TILE, OPS_PER_TILE)], didx, g1)
        a.start()
        b.start()
        a.wait()
        b.wait()

    @pl.when(wid == NW - 1)
    def _():
        a = pltpu.make_async_copy(
            src_hbm.at[pl.ds((NW - 1) * OPS_PER_TILE, LAST_REAL)],
            sidx.at[pl.ds(0, LAST_REAL)], g0)
        b = pltpu.make_async_copy(
            dst_hbm.at[pl.ds((NW - 1) * OPS_PER_TILE, LAST_REAL)],
            didx.at[pl.ds(0, LAST_REAL)], g1)
        cpad = pltpu.make_async_copy(
            pad_hbm, sidx.at[pl.ds(LAST_REAL, PAD_OPS)], s0)
        dpad = pltpu.make_async_copy(
            pad_hbm, didx.at[pl.ds(LAST_REAL, PAD_OPS)], s1)
        a.start()
        b.start()
        cpad.start()
        dpad.start()
        a.wait()
        b.wait()
        cpad.wait()
        dpad.wait()

    st_z.wait()
    st_acc.wait()
    plsc.subcore_barrier()

    bufs = (rb0, rb1, rb2, rb3)
    gsem = (g0, g1, g2, g3)
    ssem = (s0, s1, s2, s3)

    def _drain(j, sems):
        pltpu.make_async_copy(z_hbm.at[pl.ds(0, CHUNK)], bufs[j],
                              sems[j]).wait()

    def _gat(k, j):
        pltpu.async_copy(zsp.at[sidx.at[k]], bufs[j], gsem[j])

    def _scat(k, j):
        pltpu.async_copy(bufs[j], acc.at[didx.at[k]], ssem[j], add=True)

    def _step(k, j, lookahead):
        _drain(j, gsem)
        _scat(k, j)
        if lookahead:
            _drain((j + 2) % 4, ssem)
            _gat(k + 2, (j + 2) % 4)

    _gat(0, 0)
    _gat(1, 1)
    _drain(0, gsem); _scat(0, 0); _gat(2, 2)
    _drain(1, gsem); _scat(1, 1); _gat(3, 3)
    _step(2, 2, True)
    _step(3, 3, True)

    def body(b, carry):
        k = 4 * b
        _step(k, 0, True)
        _step(k + 1, 1, True)
        _step(k + 2, 2, True)
        _step(k + 3, 3, True)
        return carry

    lax.fori_loop(1, OPS_PER_TILE // 4 - 1, body, 0)

    _step(OPS_PER_TILE - 4, 0, True)
    _step(OPS_PER_TILE - 3, 1, True)
    _drain(2, gsem); _scat(OPS_PER_TILE - 2, 2)
    _drain(3, gsem); _scat(OPS_PER_TILE - 1, 3)
    _drain(0, ssem)
    _drain(1, ssem)
    _drain(2, ssem)
    _drain(3, ssem)
    plsc.subcore_barrier()

    pltpu.sync_copy(acc.at[pl.ds(r0, ROWS_PER_TILE)],
                    out_hbm.at[pl.ds(c * NP + r0, ROWS_PER_TILE)])


NPR = NP // 4
NRR = N // 4
PW = 4 * H


def _fold4(v):
    return v[:, 0:H] + v[:, H:2 * H] + v[:, 2 * H:3 * H] + v[:, 3 * H:4 * H]


def _tile4(v):
    return jnp.concatenate([v, v, v, v], axis=1)


def _proj_body(x_ref, w_ref, o_ref):
    o_ref[:NRR, :] = jnp.dot(x_ref[...], w_ref[...],
                             preferred_element_type=jnp.float32)
    o_ref[NRR:, :] = jnp.zeros((NPR - NRR, PW), jnp.float32)


def _proj(x4, w1bd):
    return pl.pallas_call(
        _proj_body,
        out_shape=jax.ShapeDtypeStruct((NPR, PW), jnp.float32),
    )(x4, w1bd)


def _bn_tail(pre, gamma_ref, beta_ref, w2bd_ref, b2_ref):
    s1 = _fold4(jnp.sum(pre, axis=0, keepdims=True)) / N
    s2 = _fold4(jnp.sum(pre * pre, axis=0, keepdims=True)) / N
    mean = _tile4(s1)
    var = _tile4(s2 - s1 * s1)
    hn = (pre - mean) * lax.rsqrt(var + 1e-5) * gamma_ref[...] + beta_ref[...]
    hn = jnp.maximum(hn, 0.0)
    h2 = jnp.dot(hn, w2bd_ref[...],
                 preferred_element_type=jnp.float32) + b2_ref[...]
    return jnp.maximum(h2, 0.0)


def _mid_body(z_ref, part_ref, b1_ref, gamma_ref, beta_ref, w2bd_ref, b2_ref,
              wnbd_ref, o_ref):
    pre = (z_ref[:NRR, :] + part_ref[:NRR, :]
           + part_ref[NPR:NPR + NRR, :] + b1_ref[...])
    h2 = _bn_tail(pre, gamma_ref, beta_ref, w2bd_ref, b2_ref)
    o_ref[:NRR, :] = jnp.dot(h2, wnbd_ref[...],
                             preferred_element_type=jnp.float32)
    o_ref[NRR:, :] = jnp.zeros((NPR - NRR, PW), jnp.float32)


def _mid(z, partp, pp):
    return pl.pallas_call(
        _mid_body,
        out_shape=jax.ShapeDtypeStruct((NPR, PW), jnp.float32),
    )(z, partp, pp["b1"], pp["gamma"], pp["beta"], pp["W2bd"], pp["b2"],
      pp["Wnbd"])


def _last_body(z_ref, part_ref, b1_ref, gamma_ref, beta_ref, w2bd_ref, b2_ref,
               hw1bd_ref, batch_ref, hb1_ref, hw2_ref, hb2_ref, o_ref):
    pre = (z_ref[:NRR, :] + part_ref[:NRR, :]
           + part_ref[NPR:NPR + NRR, :] + b1_ref[...])
    h2 = _bn_tail(pre, gamma_ref, beta_ref, w2bd_ref, b2_ref)
    hz = jnp.dot(h2, hw1bd_ref[...], preferred_element_type=jnp.float32)
    gids = lax.broadcasted_iota(jnp.int32, (G, NRR), 0)
    pooled = jnp.zeros((G, H), jnp.float32)
    for j in range(4):
        onehot = jnp.where(gids == batch_ref[j:j + 1, :NRR], 1.0, 0.0)
        pooled = pooled + jnp.dot(onehot, hz[:, j * H:(j + 1) * H],
                                  preferred_element_type=jnp.float32)
    hh = jnp.maximum(pooled + hb1_ref[...], 0.0)
    o_ref[...] = jnp.dot(hh, hw2_ref[...],
                         preferred_element_type=jnp.float32) + hb2_ref[...]


def _last(z, partp, pp, head, batchp):
    return pl.pallas_call(
        _last_body,
        out_shape=jax.ShapeDtypeStruct((G, D_OUT), jnp.float32),
    )(z, partp, pp["b1"], pp["gamma"], pp["beta"], pp["W2bd"], pp["b2"],
      jnp.kron(jnp.eye(4, dtype=jnp.float32), head["W1"]), batchp,
      head["b1"].reshape(1, H), head["W2"], head["b2"].reshape(1, D_OUT))


def _pack_params(p, w_next):
    eye4 = jnp.eye(4, dtype=jnp.float32)
    return {
        "b1": _tile4(p["b1"].reshape(1, H)),
        "gamma": _tile4(p["gamma"].reshape(1, H)),
        "beta": _tile4(p["beta"].reshape(1, H)),
        "W2bd": jnp.kron(eye4, p["W2"]),
        "b2": _tile4(p["b2"].reshape(1, H)),
        "Wnbd": None if w_next is None else jnp.kron(eye4, w_next),
    }


def kernel(x, edge_index, batch, params):
    src = edge_index[0].reshape(REAL_CHUNKS, CHUNK)
    dst = edge_index[1].reshape(REAL_CHUNKS, CHUNK)
    convs = params["convs"]
    zeros = jnp.zeros((NP, H), jnp.float32)
    pad_idx = jnp.full((PAD_OPS, CHUNK), N, jnp.int32)
    w1bd0 = jnp.kron(jnp.eye(4, dtype=jnp.float32), convs[0]["W1"])
    batchp = jnp.concatenate(
        [batch, jnp.full((NP - N,), G, jnp.int32)]).reshape(NPR, 4).T
    z = _proj(x.reshape(NRR, 4 * D_IN), w1bd0)
    for l in range(5):
        part = _sc_segment_sum(z.reshape(NP, H), src, dst, zeros, pad_idx)
        partp = part.reshape(2 * NPR, PW)
        if l < 4:
            z = _mid(z, partp, _pack_params(convs[l], convs[l + 1]["W1"]))
        else:
            out = _last(z, partp, _pack_params(convs[4], None),
                        params["head"], batchp)
    return out
